# Initial kernel scaffold; baseline (speedup 1.0000x reference)
import jax
import jax.numpy as jnp
from jax.experimental import pallas as pl


def kernel(flat1, cu_seqlens1, flat2, cu_seqlens2):
    raise NotImplementedError("write your pallas kernel here")


# speedup vs baseline: 5.2601x; 5.2601x over previous
import functools

import jax
import jax.numpy as jnp
from jax import lax
from jax.experimental import pallas as pl
from jax.experimental.pallas import tpu as pltpu
from jax.experimental.pallas import tpu_sc as plsc

_NC = 2
_NS = 16
_NW = _NC * _NS
_L = 16

_NSEG = 8
_C = 64


@functools.lru_cache(maxsize=None)
def _make_kernel(T, D):
    assert T % _NW == 0
    rows_per_w = T // _NW
    n_chunks = rows_per_w // _C
    assert rows_per_w % _C == 0

    mesh = plsc.VectorSubcoreMesh(core_axis_name="c", subcore_axis_name="s",
                                  num_cores=_NC, num_subcores=_NS)

    @functools.partial(
        pl.kernel,
        out_type=(
            jax.ShapeDtypeStruct((2 * T, D), jnp.float32),
            jax.ShapeDtypeStruct((_L,), jnp.int32),
        ),
        mesh=mesh,
        compiler_params=pltpu.CompilerParams(needs_layout_passes=False),
        scratch_types=[
            pltpu.VMEM((_L,), jnp.int32),
            pltpu.VMEM((_L,), jnp.int32),
            pltpu.VMEM((_L,), jnp.int32),
            pltpu.VMEM((_C, D), jnp.float32),
            pltpu.VMEM((_C,), jnp.int32),
            pltpu.SemaphoreType.DMA,
        ],
    )
    def k(flat1_hbm, cu1_hbm, flat2_hbm, cu2_hbm, out_hbm, cuout_hbm,
          cu1_v, cu2_v, cuout_v, dbuf, ibuf, sem):
        wid = lax.axis_index("s") * _NC + lax.axis_index("c")
        iota = lax.iota(jnp.int32, _L)

        pltpu.sync_copy(cu1_hbm, cu1_v)
        pltpu.sync_copy(cu2_hbm, cu2_v)
        cu1 = cu1_v[...]
        cu2 = cu2_v[...]

        s_cu1 = [jnp.sum(jnp.where(iota == b, cu1, 0)) for b in range(_NSEG + 1)]
        s_cu2 = [jnp.sum(jnp.where(iota == b, cu2, 0)) for b in range(_NSEG + 1)]
        len1 = [s_cu1[b + 1] - s_cu1[b] for b in range(_NSEG)]
        len2 = [s_cu2[b + 1] - s_cu2[b] for b in range(_NSEG)]
        cu_out = [jnp.int32(0)]
        for b in range(_NSEG):
            cu_out.append(cu_out[b] + len1[b] + len2[b])
        off1 = [cu_out[b] - s_cu1[b] for b in range(_NSEG + 1)]
        off2 = [cu_out[b] + (len1[b] if b < _NSEG else 0) - s_cu2[b]
                for b in range(_NSEG + 1)]

        cuvec = jnp.zeros((_L,), jnp.int32)
        for b in range(_NSEG + 1):
            cuvec = jnp.where(iota == b, cu_out[b], cuvec)

        @pl.when(wid == 0)
        def _():
            cuout_v[...] = cuvec
            pltpu.sync_copy(cuout_v, cuout_hbm)

        def do_source(src_hbm, s_cu, off):
            base_w = wid * rows_per_w
            for c in range(n_chunks):
                base = base_w + c * _C
                pltpu.sync_copy(src_hbm.at[pl.ds(base, _C)], dbuf)
                for j in range(_C // _L):
                    t = base + j * _L + iota
                    d = t + off[0]
                    for b in range(1, _NSEG + 1):
                        d = d + jnp.where(t >= s_cu[b], off[b] - off[b - 1], 0)
                    ibuf[pl.ds(j * _L, _L)] = d
                pltpu.async_copy(dbuf, out_hbm.at[ibuf], sem).wait()

        do_source(flat1_hbm, s_cu1, off1)
        do_source(flat2_hbm, s_cu2, off2)

    return k


def kernel(flat1, cu_seqlens1, flat2, cu_seqlens2):
    T, D = flat1.shape
    k = _make_kernel(T, D)
    pad = jnp.full((_L - cu_seqlens1.shape[0],), T, jnp.int32)
    cu1p = jnp.concatenate([cu_seqlens1.astype(jnp.int32), pad])
    cu2p = jnp.concatenate([cu_seqlens2.astype(jnp.int32), pad])
    out, cu_out_pad = k(flat1, cu1p, flat2, cu2p)
    return out, cu_out_pad[: cu_seqlens1.shape[0]]


# speedup vs baseline: 5.6319x; 1.0707x over previous
import functools

import jax
import jax.numpy as jnp
from jax import lax
from jax.experimental import pallas as pl
from jax.experimental.pallas import tpu as pltpu
from jax.experimental.pallas import tpu_sc as plsc

_NC = 2
_NS = 16
_NW = _NC * _NS
_L = 16

_NSEG = 8
_C = 32
_NBUF = 3


@functools.lru_cache(maxsize=None)
def _make_kernel(T, D):
    assert T % _NW == 0
    rows_per_w = T // _NW
    n_chunks = rows_per_w // _C
    assert rows_per_w % _C == 0

    mesh = plsc.VectorSubcoreMesh(core_axis_name="c", subcore_axis_name="s",
                                  num_cores=_NC, num_subcores=_NS)

    @functools.partial(
        pl.kernel,
        out_type=(
            jax.ShapeDtypeStruct((2 * T, D), jnp.float32),
            jax.ShapeDtypeStruct((_L,), jnp.int32),
        ),
        mesh=mesh,
        compiler_params=pltpu.CompilerParams(needs_layout_passes=False),
        scratch_types=[
            pltpu.VMEM((_L,), jnp.int32),
            pltpu.VMEM((_L,), jnp.int32),
            pltpu.VMEM((_L,), jnp.int32),
            [pltpu.VMEM((_C, D), jnp.float32)] * _NBUF,
            [pltpu.VMEM((_C,), jnp.int32)] * _NBUF,
            [pltpu.SemaphoreType.DMA] * _NBUF,
            [pltpu.SemaphoreType.DMA] * _NBUF,
        ],
    )
    def k(flat1_hbm, cu1_hbm, flat2_hbm, cu2_hbm, out_hbm, cuout_hbm,
          cu1_v, cu2_v, cuout_v, dbufs, ibufs, semL, semS):
        wid = lax.axis_index("s") * _NC + lax.axis_index("c")
        iota = lax.iota(jnp.int32, _L)

        base_w = wid * rows_per_w
        chunks = ([(flat1_hbm, base_w + c * _C, 0) for c in range(n_chunks)]
                  + [(flat2_hbm, base_w + c * _C, 1) for c in range(n_chunks)])
        nch = len(chunks)

        def start_load(c):
            ref, base, _ = chunks[c]
            i = c % _NBUF
            return pltpu.async_copy(ref.at[pl.ds(base, _C)], dbufs[i], semL[i])

        loadd = {0: start_load(0), 1: start_load(1)}

        pltpu.sync_copy(cu1_hbm, cu1_v)
        pltpu.sync_copy(cu2_hbm, cu2_v)
        cu1 = cu1_v[...]
        cu2 = cu2_v[...]

        s_cu1 = [jnp.sum(jnp.where(iota == b, cu1, 0)) for b in range(_NSEG + 1)]
        s_cu2 = [jnp.sum(jnp.where(iota == b, cu2, 0)) for b in range(_NSEG + 1)]
        len1 = [s_cu1[b + 1] - s_cu1[b] for b in range(_NSEG)]
        len2 = [s_cu2[b + 1] - s_cu2[b] for b in range(_NSEG)]
        cu_out = [jnp.int32(0)]
        for b in range(_NSEG):
            cu_out.append(cu_out[b] + len1[b] + len2[b])
        off1 = [cu_out[b] - s_cu1[b] for b in range(_NSEG + 1)]
        off2 = [cu_out[b] + (len1[b] if b < _NSEG else 0) - s_cu2[b]
                for b in range(_NSEG + 1)]

        cuvec = jnp.zeros((_L,), jnp.int32)
        for b in range(_NSEG + 1):
            cuvec = jnp.where(iota == b, cu_out[b], cuvec)

        @pl.when(wid == 0)
        def _():
            cuout_v[...] = cuvec
            pltpu.sync_copy(cuout_v, cuout_hbm)

        scatd = {}
        waited = set()
        for c in range(nch):
            i = c % _NBUF
            loadd[c].wait()
            _, base, which = chunks[c]
            s_cu, off = (s_cu1, off1) if which == 0 else (s_cu2, off2)
            for j in range(_C // _L):
                t = base + j * _L + iota
                d = t + off[0]
                for b in range(1, _NSEG + 1):
                    d = d + jnp.where(t >= s_cu[b], off[b] - off[b - 1], 0)
                ibufs[i][pl.ds(j * _L, _L)] = d
            scatd[c] = pltpu.async_copy(dbufs[i], out_hbm.at[ibufs[i]], semS[i])
            if c + 2 < nch:
                if c >= 1:
                    scatd[c - 1].wait()
                    waited.add(c - 1)
                loadd[c + 2] = start_load(c + 2)
        for c in range(nch):
            if c not in waited:
                scatd[c].wait()

    return k


def kernel(flat1, cu_seqlens1, flat2, cu_seqlens2):
    T, D = flat1.shape
    k = _make_kernel(T, D)
    pad = jnp.full((_L - cu_seqlens1.shape[0],), T, jnp.int32)
    cu1p = jnp.concatenate([cu_seqlens1.astype(jnp.int32), pad])
    cu2p = jnp.concatenate([cu_seqlens2.astype(jnp.int32), pad])
    out, cu_out_pad = k(flat1, cu1p, flat2, cu2p)
    return out, cu_out_pad[: cu_seqlens1.shape[0]]
